# Initial kernel scaffold; baseline (speedup 1.0000x reference)
import jax
import jax.numpy as jnp
from jax.experimental import pallas as pl


def kernel(x, W1, b1, W2, b2, W3, b3):
    raise NotImplementedError("write your pallas kernel here")


# speedup vs baseline: 22.7248x; 22.7248x over previous
import jax
import jax.numpy as jnp
from jax.experimental import pallas as pl
from jax.experimental.pallas import tpu as pltpu

_B = 8
_N = 32768
_S = 1024


def _fps_kernel(x0_ref, x1_ref, x2_ref, cinit_ref, pts_ref, dist_ref):
    dist_ref[...] = jnp.full((_B, _N), 1e10, jnp.float32)
    lane = jax.lax.broadcasted_iota(jnp.int32, (_B, _N), 1)

    def body(t, c):
        c0, c1, c2 = c
        pts_ref[t] = jnp.concatenate([c0, c1, c2], axis=1)
        d0 = x0_ref[...] - c0
        d1 = x1_ref[...] - c1
        d2 = x2_ref[...] - c2
        d = d0 * d0 + d1 * d1 + d2 * d2
        dist = jnp.minimum(dist_ref[...], d)
        dist_ref[...] = dist
        m = jnp.max(dist, axis=1, keepdims=True)
        cand = jnp.where(dist == m, lane, _N)
        idx = jnp.min(cand, axis=1, keepdims=True)
        sel = lane == idx
        n0 = jnp.sum(jnp.where(sel, x0_ref[...], 0.0), axis=1, keepdims=True)
        n1 = jnp.sum(jnp.where(sel, x1_ref[...], 0.0), axis=1, keepdims=True)
        n2 = jnp.sum(jnp.where(sel, x2_ref[...], 0.0), axis=1, keepdims=True)
        return (n0, n1, n2)

    c0 = cinit_ref[:, 0:1]
    c1 = cinit_ref[:, 1:2]
    c2 = cinit_ref[:, 2:3]
    jax.lax.fori_loop(0, _S, body, (c0, c1, c2))


def _mlp_kernel(p_ref, w1_ref, b1_ref, w2_ref, b2_ref, w3_ref, b3_ref, out_ref):
    p = p_ref[...]
    h = jnp.dot(p, w1_ref[...], preferred_element_type=jnp.float32)
    h = jnp.maximum(h + b1_ref[...], 0.0)
    h = jnp.dot(h, w2_ref[...], preferred_element_type=jnp.float32)
    h = jnp.maximum(h + b2_ref[...], 0.0)
    h = jnp.dot(h, w3_ref[...], preferred_element_type=jnp.float32)
    out_ref[...] = h + b3_ref[...]


def kernel(x, W1, b1, W2, b2, W3, b3):
    B, N, _ = x.shape
    init_idx = jax.random.randint(jax.random.key(1), (B,), 0, N, dtype=jnp.int32)
    cinit = x[jnp.arange(B), init_idx, :]

    x0 = x[:, :, 0]
    x1 = x[:, :, 1]
    x2 = x[:, :, 2]

    pts = pl.pallas_call(
        _fps_kernel,
        out_shape=jax.ShapeDtypeStruct((_S, B, 3), jnp.float32),
        scratch_shapes=[pltpu.VMEM((_B, _N), jnp.float32)],
    )(x0, x1, x2, cinit)

    sampled = jnp.transpose(pts, (1, 0, 2))

    feats = pl.pallas_call(
        _mlp_kernel,
        out_shape=jax.ShapeDtypeStruct((B * _S, 256), jnp.float32),
    )(
        sampled.reshape(B * _S, 3),
        W1,
        b1.reshape(1, 64),
        W2,
        b2.reshape(1, 128),
        W3,
        b3.reshape(1, 256),
    )

    return sampled, feats.reshape(B, _S, 256)
